# Initial kernel scaffold; baseline (speedup 1.0000x reference)
import jax
import jax.numpy as jnp
from jax.experimental import pallas as pl


def kernel(x, edge_attr, edge_index, mp_fc0_w, mp_fc0_b, mp_out_w, mp_out_b, fc0_w, fc0_b, fc_out_w, fc_out_b):
    raise NotImplementedError("write your pallas kernel here")


# speedup vs baseline: 2.6730x; 2.6730x over previous
import jax
import jax.numpy as jnp
from jax import lax
from jax.experimental import pallas as pl
from jax.experimental.pallas import tpu as pltpu
from jax.experimental.pallas import tpu_sc as plsc

N_NODES = 10000
N_EDGES = 320000
D_FEAT = 128
HALF = 144

NC = 2
NS = 16

N_PAD = 10240
E_PAD = 327680

CH = 80
IB = 32
N_CHUNKS = E_PAD // CH
SC1_CHUNKS = N_CHUNKS // NS
SC1_BLOCKS = SC1_CHUNKS // IB
SC2_CHUNKS = N_CHUNKS // (NC * NS)
SC2_BLOCKS = SC2_CHUNKS // IB
STRIPE = N_PAD // NS

MB = 1024
EB = 4000


def _dense_pre_body(x_ref, w1_ref, b1_ref, w2_ref, b2_ref,
                    wcat_a_ref, wcat_b_ref, wx_a_ref, wx_b_ref, bias_b_ref,
                    h2a_ref, h2b_ref, xwa_ref, xwb_ref):
    x = x_ref[...]
    h1 = jnp.maximum(x @ w1_ref[...] + b1_ref[...], 0.0)
    h = h1 @ w2_ref[...] + b2_ref[...]
    h2a_ref[...] = h @ wcat_a_ref[...]
    h2b_ref[...] = h @ wcat_b_ref[...]
    xwa_ref[...] = x @ wx_a_ref[...]
    xwb_ref[...] = x @ wx_b_ref[...] + bias_b_ref[...]


def _dense_pre(x, w1, b1, w2, b2, wcat_a, wcat_b, wx_a, wx_b, bias_b):
    grid = (N_PAD // MB,)
    full = lambda shape: pl.BlockSpec(shape, lambda i: (0, 0))
    return pl.pallas_call(
        _dense_pre_body,
        grid=grid,
        in_specs=[
            pl.BlockSpec((MB, D_FEAT), lambda i: (i, 0)),
            full((D_FEAT, 128)), full((1, 128)),
            full((128, 512)), full((1, 512)),
            full((512, HALF)), full((512, HALF)),
            full((D_FEAT, HALF)), full((D_FEAT, HALF)), full((1, HALF)),
        ],
        out_specs=[pl.BlockSpec((MB, HALF), lambda i: (i, 0))] * 4,
        out_shape=[jax.ShapeDtypeStruct((N_PAD, HALF), jnp.float32)] * 4,
    )(x, w1, b1, w2, b2, wcat_a, wcat_b, wx_a, wx_b, bias_b)


def _sc_segsum_body(h2a_hbm, h2b_hbm, xwa_hbm, xwb_hbm, src_hbm, dst_hbm,
                    pa_hbm, pb_hbm,
                    idx_s, idx_d, rows, sem, acc):
    c = lax.axis_index("c")
    s = lax.axis_index("s")

    @pl.when(c == 0)
    def _():
        pltpu.sync_copy(xwa_hbm.at[pl.ds(s * STRIPE, STRIPE)],
                        acc.at[pl.ds(s * STRIPE, STRIPE)])

    @pl.when(c == 1)
    def _():
        pltpu.sync_copy(xwb_hbm.at[pl.ds(s * STRIPE, STRIPE)],
                        acc.at[pl.ds(s * STRIPE, STRIPE)])

    plsc.subcore_barrier()

    def block(g, _):
        row0 = s * SC1_CHUNKS + g * IB
        pltpu.sync_copy(src_hbm.at[pl.ds(row0, IB)], idx_s)
        pltpu.sync_copy(dst_hbm.at[pl.ds(row0, IB)], idx_d)

        def chunk(j, _):
            @pl.when(c == 0)
            def _():
                pltpu.async_copy(h2a_hbm.at[idx_s.at[j]], rows, sem).wait()

            @pl.when(c == 1)
            def _():
                pltpu.async_copy(h2b_hbm.at[idx_s.at[j]], rows, sem).wait()

            pltpu.sync_copy(rows, acc.at[idx_d.at[j]], add=True)
            return 0

        lax.fori_loop(0, IB, chunk, 0)
        return 0

    lax.fori_loop(0, SC1_BLOCKS, block, 0)
    plsc.subcore_barrier()

    @pl.when(c == 0)
    def _():
        pltpu.sync_copy(acc.at[pl.ds(s * STRIPE, STRIPE)],
                        pa_hbm.at[pl.ds(s * STRIPE, STRIPE)])

    @pl.when(c == 1)
    def _():
        pltpu.sync_copy(acc.at[pl.ds(s * STRIPE, STRIPE)],
                        pb_hbm.at[pl.ds(s * STRIPE, STRIPE)])


def _sc_segsum(h2a, h2b, xwa, xwb, src2d, dst2d):
    mesh = plsc.VectorSubcoreMesh(core_axis_name="c", subcore_axis_name="s",
                                  num_cores=NC, num_subcores=NS)
    return pl.kernel(
        _sc_segsum_body,
        compiler_params=pltpu.CompilerParams(use_tc_tiling_on_sc=False),
        out_type=[jax.ShapeDtypeStruct((N_PAD, HALF), jnp.float32)] * 2,
        mesh=mesh,
        scratch_types=[
            pltpu.VMEM((IB, CH), jnp.int32),
            pltpu.VMEM((IB, CH), jnp.int32),
            pltpu.VMEM((CH, HALF), jnp.float32),
            pltpu.SemaphoreType.DMA,
            pltpu.VMEM_SHARED((N_PAD, HALF), jnp.float32),
        ],
    )(h2a, h2b, xwa, xwb, src2d, dst2d)


def _sc_edge_body(pa_hbm, pb_hbm, src_hbm, dst_hbm, t0_hbm, t1_hbm,
                  idx_s, idx_d, rows0, rows1, sem0, sem1):
    c = lax.axis_index("c")
    s = lax.axis_index("s")
    wid = c * NS + s

    def block(g, _):
        row0 = wid * SC2_CHUNKS + g * IB
        pltpu.sync_copy(src_hbm.at[pl.ds(row0, IB)], idx_s)
        pltpu.sync_copy(dst_hbm.at[pl.ds(row0, IB)], idx_d)

        def chunk(j, _):
            cp0 = pltpu.async_copy(pa_hbm.at[idx_s.at[j]], rows0, sem0)
            cp1 = pltpu.async_copy(pb_hbm.at[idx_d.at[j]], rows1, sem1)
            cp0.wait()
            cp1.wait()
            off = (row0 + j) * CH
            pltpu.sync_copy(rows0, t0_hbm.at[pl.ds(off, CH)])
            pltpu.sync_copy(rows1, t1_hbm.at[pl.ds(off, CH)])
            return 0

        lax.fori_loop(0, IB, chunk, 0)
        return 0

    lax.fori_loop(0, SC2_BLOCKS, block, 0)


def _sc_edge(pa, pb, src2d, dst2d):
    mesh = plsc.VectorSubcoreMesh(core_axis_name="c", subcore_axis_name="s",
                                  num_cores=NC, num_subcores=NS)
    return pl.kernel(
        _sc_edge_body,
        compiler_params=pltpu.CompilerParams(use_tc_tiling_on_sc=False),
        out_type=[jax.ShapeDtypeStruct((E_PAD, HALF), jnp.float32)] * 2,
        mesh=mesh,
        scratch_types=[
            pltpu.VMEM((IB, CH), jnp.int32),
            pltpu.VMEM((IB, CH), jnp.int32),
            pltpu.VMEM((CH, HALF), jnp.float32),
            pltpu.VMEM((CH, HALF), jnp.float32),
            pltpu.SemaphoreType.DMA,
            pltpu.SemaphoreType.DMA,
        ],
    )(pa, pb, src2d, dst2d)


def _final_body(t0_ref, t1_ref, vh_ref, o_ref):
    t0 = t0_ref[...]
    t1 = t1_ref[...]
    hid = jnp.maximum(t0[:, :128] + t1[:, :128], 0.0)
    o_ref[...] = hid @ vh_ref[...] + t0[:, 128:] + t1[:, 128:]


def _final(t0, t1, vh):
    grid = (N_EDGES // EB,)
    return pl.pallas_call(
        _final_body,
        grid=grid,
        in_specs=[
            pl.BlockSpec((EB, HALF), lambda i: (i, 0)),
            pl.BlockSpec((EB, HALF), lambda i: (i, 0)),
            pl.BlockSpec((128, 16), lambda i: (0, 0)),
        ],
        out_specs=pl.BlockSpec((EB, 16), lambda i: (i, 0)),
        out_shape=jax.ShapeDtypeStruct((N_EDGES, 16), jnp.float32),
    )(t0, t1, vh)


def kernel(x, edge_attr, edge_index, mp_fc0_w, mp_fc0_b, mp_out_w, mp_out_b,
           fc0_w, fc0_b, fc_out_w, fc_out_b):
    del edge_attr

    src = edge_index[0].astype(jnp.int32)
    dst = edge_index[1].astype(jnp.int32)
    pad_e = E_PAD - N_EDGES
    src2d = jnp.concatenate(
        [src, jnp.zeros((pad_e,), jnp.int32)]).reshape(N_CHUNKS, CH)
    dst2d = jnp.concatenate(
        [dst, jnp.full((pad_e,), N_NODES, jnp.int32)]).reshape(N_CHUNKS, CH)

    x_pad = jnp.pad(x, ((0, N_PAD - N_NODES), (0, 0)))

    W_xs = fc0_w[:, 0:128].T
    W_xd = fc0_w[:, 128:256].T
    W_as = fc0_w[:, 256:768].T
    W_ad = fc0_w[:, 768:1280].T
    V_h = fc_out_w[:, 0:128].T
    V_xs = fc_out_w[:, 128:256].T
    V_xd = fc_out_w[:, 256:384].T
    V_as = fc_out_w[:, 384:896].T
    V_ad = fc_out_w[:, 896:1408].T

    wcat_a = jnp.concatenate([W_as, V_as], axis=1)
    wcat_b = jnp.concatenate([W_ad, V_ad], axis=1)
    wx_a = jnp.concatenate([W_xs, V_xs], axis=1)
    wx_b = jnp.concatenate([W_xd, V_xd], axis=1)
    bias_b = jnp.concatenate([fc0_b, fc_out_b])[None, :]

    h2a, h2b, xwa, xwb = _dense_pre(
        x_pad, mp_fc0_w.T, mp_fc0_b[None, :], mp_out_w.T, mp_out_b[None, :],
        wcat_a, wcat_b, wx_a, wx_b, bias_b)

    pa, pb = _sc_segsum(h2a, h2b, xwa, xwb, src2d, dst2d)
    t0, t1 = _sc_edge(pa, pb, src2d, dst2d)
    return _final(t0, t1, V_h)
